# baseline (speedup 1.0000x reference)
import jax
import jax.numpy as jnp
from jax import lax
from jax.experimental import pallas as pl
from jax.experimental.pallas import tpu as pltpu
from jax.experimental.pallas import tpu_sc as plsc

N_NODES = 50000
N_EDGES = 800000
D_IN = 100
HIDDEN = 256
NQ = 4
DQ = 25
DQP = 32
DOUT = 128
N_SC = 2
N_TILES = 16
STRIPE = 3200
LAST_STRIPE = N_NODES - (N_TILES - 1) * STRIPE
EB = 128
BLK_PER_TILE = 392
NBLK = BLK_PER_TILE * N_TILES
E_PAD = NBLK * EB
ACC_ROWS = 50048
NGRP = BLK_PER_TILE // 4
EB2 = 800
NBLK2 = N_EDGES // EB2
NW = N_SC * N_TILES
BLK2_PER_W = NBLK2 // NW
RB = 1000


def _stripe_copy(s, read, write):
    off = pl.multiple_of(s * STRIPE, STRIPE)

    @pl.when(s < N_TILES - 1)
    def _main():
        pltpu.sync_copy(read(pl.ds(off, STRIPE)), write(pl.ds(off, STRIPE)))

    @pl.when(s == N_TILES - 1)
    def _last():
        base = (N_TILES - 1) * STRIPE
        pltpu.sync_copy(read(pl.ds(base, LAST_STRIPE)),
                        write(pl.ds(base, LAST_STRIPE)))


def _agg_body(xf_hbm, src_hbm, dst_hbm, out_hbm, tab_hbm, acc_sh, sidx_a,
              sidx_b, didx_a, didx_b, rows_a, rows_b, isem_a, isem_b, gsem_a,
              gsem_b):
    c = lax.axis_index("c")
    s = lax.axis_index("s")

    for cc in range(N_SC):
        for pp in range(2):
            q = 2 * cc + pp

            @pl.when(c == cc)
            def _mktab(q=q):
                _stripe_copy(s, lambda d: xf_hbm.at[d, pl.ds(q * DQP, DQP)],
                             lambda d: tab_hbm.at[q, d])

    plsc.subcore_barrier()

    for p in range(2):
        for cc in range(N_SC):
            q = 2 * cc + p

            @pl.when(c == cc)
            def _init(q=q):
                _stripe_copy(s, lambda d: tab_hbm.at[q, d],
                             lambda d: acc_sh.at[d])

        plsc.subcore_barrier()

        for cc in range(N_SC):
            q = 2 * cc + p

            @pl.when(c == cc)
            def _edges(q=q):
                table = tab_hbm.at[q]
                base = pl.multiple_of(s * BLK_PER_TILE, BLK_PER_TILE)

                def load(off, sidx, didx, sem):
                    d = pl.ds(pl.multiple_of(off, 2), 2)
                    pltpu.async_copy(src_hbm.at[d], sidx, sem)
                    return pltpu.async_copy(dst_hbm.at[d], didx, sem)

                def load_wait(off, sidx, didx, sem):
                    d = pl.ds(pl.multiple_of(off, 2), 2)
                    pltpu.make_async_copy(src_hbm.at[d], sidx, sem).wait()
                    pltpu.make_async_copy(dst_hbm.at[d], didx, sem).wait()

                def gather(j, sidx, rows, sem):
                    return pltpu.async_copy(table.at[sidx.at[j]],
                                            rows.at[j], sem)

                def gather_wait(j, sidx, rows, sem):
                    pltpu.make_async_copy(table.at[sidx.at[j]],
                                          rows.at[j], sem).wait()

                def scat(j, didx, rows):
                    pltpu.sync_copy(rows.at[j], acc_sh.at[didx.at[j]],
                                    add=True)

                load(base, sidx_a, didx_a, isem_a)
                load(base + 2, sidx_b, didx_b, isem_b)
                load_wait(base, sidx_a, didx_a, isem_a)
                gather(0, sidx_a, rows_a, gsem_a)
                gather(1, sidx_a, rows_a, gsem_a)

                def grp(g, carry):
                    koff = pl.multiple_of(base + 4 * g, 2)
                    gather_wait(0, sidx_a, rows_a, gsem_a)
                    gather_wait(1, sidx_a, rows_a, gsem_a)
                    load_wait(koff + 2, sidx_b, didx_b, isem_b)
                    hb0 = gather(0, sidx_b, rows_b, gsem_b)
                    hb1 = gather(1, sidx_b, rows_b, gsem_b)
                    scat(0, didx_a, rows_a)
                    scat(1, didx_a, rows_a)
                    hla = load(koff + 4, sidx_a, didx_a, isem_a)
                    hb0.wait()
                    hb1.wait()
                    hla.wait()
                    pltpu.make_async_copy(
                        src_hbm.at[pl.ds(pl.multiple_of(koff + 4, 2), 2)],
                        sidx_a, isem_a).wait()
                    gather(0, sidx_a, rows_a, gsem_a)
                    gather(1, sidx_a, rows_a, gsem_a)
                    scat(0, didx_b, rows_b)
                    scat(1, didx_b, rows_b)
                    load(koff + 6, sidx_b, didx_b, isem_b)
                    return carry

                lax.fori_loop(0, NGRP - 1, grp, 0)

                gather_wait(0, sidx_a, rows_a, gsem_a)
                gather_wait(1, sidx_a, rows_a, gsem_a)
                load_wait(base + BLK_PER_TILE - 2, sidx_b, didx_b, isem_b)
                hb0 = gather(0, sidx_b, rows_b, gsem_b)
                hb1 = gather(1, sidx_b, rows_b, gsem_b)
                scat(0, didx_a, rows_a)
                scat(1, didx_a, rows_a)
                hb0.wait()
                hb1.wait()
                scat(0, didx_b, rows_b)
                scat(1, didx_b, rows_b)

        plsc.subcore_barrier()

        for cc in range(N_SC):
            q = 2 * cc + p

            @pl.when(c == cc)
            def _flush(q=q):
                _stripe_copy(s, lambda d: acc_sh.at[d],
                             lambda d: out_hbm.at[d, pl.ds(q * DQP, DQP)])

        plsc.subcore_barrier()


_agg = pl.kernel(
    _agg_body,
    out_type=(jax.ShapeDtypeStruct((N_NODES, DOUT), jnp.float32),
              jax.ShapeDtypeStruct((NQ, N_NODES, DQP), jnp.float32)),
    mesh=plsc.VectorSubcoreMesh(core_axis_name="c", subcore_axis_name="s"),
    scratch_types=[
        pltpu.VMEM_SHARED((ACC_ROWS, DQP), jnp.float32),
        pltpu.VMEM((2, EB), jnp.int32),
        pltpu.VMEM((2, EB), jnp.int32),
        pltpu.VMEM((2, EB), jnp.int32),
        pltpu.VMEM((2, EB), jnp.int32),
        pltpu.VMEM((2, EB, DQP), jnp.float32),
        pltpu.VMEM((2, EB, DQP), jnp.float32),
        pltpu.SemaphoreType.DMA,
        pltpu.SemaphoreType.DMA,
        pltpu.SemaphoreType.DMA,
        pltpu.SemaphoreType.DMA,
    ],
    compiler_params=pltpu.CompilerParams(use_tc_tiling_on_sc=False),
)


def _mlp_body(h_ref, w1_ref, b1_ref, w2_ref, b2_ref, out_ref):
    z = lax.dot_general(h_ref[...], w1_ref[...], (((1,), (0,)), ((), ())),
                        preferred_element_type=jnp.float32,
                        precision=lax.Precision.HIGHEST)
    z = jnp.maximum(z + b1_ref[...], 0.0)
    w2s = jnp.sum(w2_ref[...], axis=1)
    out_ref[...] = (jnp.sum(z * w2s[None, :], axis=1, keepdims=True)
                    + jnp.sum(b2_ref[...]))


_mlp = pl.pallas_call(
    _mlp_body,
    grid=(N_NODES // RB,),
    in_specs=[
        pl.BlockSpec((RB, DOUT), lambda i: (i, 0)),
        pl.BlockSpec((DOUT, HIDDEN), lambda i: (0, 0)),
        pl.BlockSpec((1, HIDDEN), lambda i: (0, 0)),
        pl.BlockSpec((HIDDEN, HIDDEN), lambda i: (0, 0)),
        pl.BlockSpec((1, HIDDEN), lambda i: (0, 0)),
    ],
    out_specs=pl.BlockSpec((RB, 1), lambda i: (i, 0)),
    out_shape=jax.ShapeDtypeStruct((N_NODES, 1), jnp.float32),
)


def _gather_body(s_hbm, dst_hbm, out_hbm, s_v, dst_v, out_v):
    c = lax.axis_index("c")
    s = lax.axis_index("s")
    w = s * N_SC + c
    pltpu.sync_copy(s_hbm, s_v)
    nb = BLK2_PER_W + jnp.where(w < NBLK2 % NW, 1, 0)

    def blk(i, carry):
        off = pl.multiple_of((w + i * NW) * EB2, EB2)
        pltpu.sync_copy(dst_hbm.at[pl.ds(off, EB2)], dst_v)

        def inner(j, c2):
            idx = dst_v[pl.ds(j * 16, 16)]
            out_v[pl.ds(j * 16, 16)] = plsc.load_gather(s_v, [idx])
            return c2

        lax.fori_loop(0, EB2 // 16, inner, 0)
        pltpu.sync_copy(out_v, out_hbm.at[pl.ds(off, EB2)])
        return carry

    lax.fori_loop(0, nb, blk, 0)


_gather = pl.kernel(
    _gather_body,
    out_type=jax.ShapeDtypeStruct((N_EDGES,), jnp.float32),
    mesh=plsc.VectorSubcoreMesh(core_axis_name="c", subcore_axis_name="s"),
    scratch_types=[
        pltpu.VMEM((N_NODES,), jnp.float32),
        pltpu.VMEM((EB2,), jnp.int32),
        pltpu.VMEM((EB2,), jnp.float32),
    ],
    compiler_params=pltpu.CompilerParams(needs_layout_passes=False),
)


def kernel(x, edge_index, pos_embeddings, W1, b1, W2, b2):
    xf = jnp.pad(x.reshape(N_NODES, NQ, DQ),
                 ((0, 0), (0, 0), (0, DQP - DQ))).reshape(N_NODES, DOUT)
    ei = edge_index.astype(jnp.int32)
    srcp = jnp.concatenate(
        [ei[0], jnp.zeros((E_PAD - N_EDGES,), jnp.int32)]).reshape(NBLK, EB)
    dstp = jnp.concatenate(
        [ei[1], jnp.full((E_PAD - N_EDGES,), N_NODES,
                         jnp.int32)]).reshape(NBLK, EB)
    h, _tab = _agg(xf, srcp, dstp)
    W1p = jnp.pad(W1.reshape(NQ, DQ, HIDDEN),
                  ((0, 0), (0, DQP - DQ), (0, 0))).reshape(DOUT, HIDDEN)
    s = _mlp(h, W1p, b1.reshape(1, HIDDEN), W2,
             b2.reshape(1, HIDDEN)).reshape(N_NODES)
    return _gather(s, ei[1])


# speedup vs baseline: 1.5418x; 1.5418x over previous
import jax
import jax.numpy as jnp
from jax import lax
from jax.experimental import pallas as pl
from jax.experimental.pallas import tpu as pltpu
from jax.experimental.pallas import tpu_sc as plsc

N_NODES = 50000
N_EDGES = 800000
D_IN = 100
HIDDEN = 256
NQ = 4
DQ = 25
DQP = 32
DOUT = NQ * DQP
N_SC = 2
N_TILES = 16
STRIPE = 3200
LAST_STRIPE = N_NODES - (N_TILES - 1) * STRIPE
EB = 128
BLK_PER_TILE = 392
NBLK = BLK_PER_TILE * N_TILES
E_PAD = NBLK * EB
ACC_ROWS = 50048
NGRP = BLK_PER_TILE // 4
EB2 = 800
NBLK2 = N_EDGES // EB2
NW = N_SC * N_TILES
BLK2_PER_W = NBLK2 // NW
RB = 1000


def _stripe_copy(s, read, write):
    off = pl.multiple_of(s * STRIPE, STRIPE)

    @pl.when(s < N_TILES - 1)
    def _main():
        pltpu.sync_copy(read(pl.ds(off, STRIPE)), write(pl.ds(off, STRIPE)))

    @pl.when(s == N_TILES - 1)
    def _last():
        base = (N_TILES - 1) * STRIPE
        pltpu.sync_copy(read(pl.ds(base, LAST_STRIPE)),
                        write(pl.ds(base, LAST_STRIPE)))


def _agg_body(xq_hbm, edges_hbm, out_hbm, acc_sh, idx_a, idx_b, rows_a,
              rows_b, isem_a, isem_b, gsem_a, gsem_b):
    c = lax.axis_index("c")
    s = lax.axis_index("s")

    for p in range(2):
        for cc in range(N_SC):
            q = 2 * cc + p

            @pl.when(c == cc)
            def _init(q=q):
                _stripe_copy(s, lambda d: xq_hbm.at[q, d],
                             lambda d: acc_sh.at[d])

        plsc.subcore_barrier()

        for cc in range(N_SC):
            q = 2 * cc + p

            @pl.when(c == cc)
            def _edges(q=q):
                table = xq_hbm.at[q]
                base = pl.multiple_of(s * BLK_PER_TILE, BLK_PER_TILE)

                def idx_slice(off):
                    return edges_hbm.at[pl.ds(pl.multiple_of(off, 2), 2)]

                def gather(j, idx, rows, sem):
                    return pltpu.async_copy(table.at[idx.at[j, 0]],
                                            rows.at[j], sem)

                def gather_wait(j, idx, rows, sem):
                    pltpu.make_async_copy(table.at[idx.at[j, 0]],
                                          rows.at[j], sem).wait()

                def scat(j, idx, rows):
                    pltpu.sync_copy(rows.at[j], acc_sh.at[idx.at[j, 1]],
                                    add=True)

                pltpu.async_copy(idx_slice(base), idx_a, isem_a)
                pltpu.async_copy(idx_slice(base + 2), idx_b, isem_b)
                pltpu.make_async_copy(idx_slice(base), idx_a, isem_a).wait()
                gather(0, idx_a, rows_a, gsem_a)
                gather(1, idx_a, rows_a, gsem_a)

                def grp(g, carry):
                    koff = pl.multiple_of(base + 4 * g, 2)
                    gather_wait(0, idx_a, rows_a, gsem_a)
                    gather_wait(1, idx_a, rows_a, gsem_a)
                    pltpu.make_async_copy(idx_slice(koff + 2), idx_b,
                                          isem_b).wait()
                    hb0 = gather(0, idx_b, rows_b, gsem_b)
                    hb1 = gather(1, idx_b, rows_b, gsem_b)
                    scat(0, idx_a, rows_a)
                    scat(1, idx_a, rows_a)
                    hla = pltpu.async_copy(idx_slice(koff + 4), idx_a, isem_a)
                    hb0.wait()
                    hb1.wait()
                    hla.wait()
                    gather(0, idx_a, rows_a, gsem_a)
                    gather(1, idx_a, rows_a, gsem_a)
                    scat(0, idx_b, rows_b)
                    scat(1, idx_b, rows_b)
                    pltpu.async_copy(idx_slice(koff + 6), idx_b, isem_b)
                    return carry

                lax.fori_loop(0, NGRP - 1, grp, 0)

                gather_wait(0, idx_a, rows_a, gsem_a)
                gather_wait(1, idx_a, rows_a, gsem_a)
                pltpu.make_async_copy(idx_slice(base + BLK_PER_TILE - 2),
                                      idx_b, isem_b).wait()
                hb0 = gather(0, idx_b, rows_b, gsem_b)
                hb1 = gather(1, idx_b, rows_b, gsem_b)
                scat(0, idx_a, rows_a)
                scat(1, idx_a, rows_a)
                hb0.wait()
                hb1.wait()
                scat(0, idx_b, rows_b)
                scat(1, idx_b, rows_b)

        plsc.subcore_barrier()

        for cc in range(N_SC):
            q = 2 * cc + p

            @pl.when(c == cc)
            def _flush(q=q):
                _stripe_copy(s, lambda d: acc_sh.at[d],
                             lambda d: out_hbm.at[q, d])

        plsc.subcore_barrier()


_agg = pl.kernel(
    _agg_body,
    out_type=jax.ShapeDtypeStruct((NQ, N_NODES, DQP), jnp.float32),
    mesh=plsc.VectorSubcoreMesh(core_axis_name="c", subcore_axis_name="s"),
    scratch_types=[
        pltpu.VMEM_SHARED((ACC_ROWS, DQP), jnp.float32),
        pltpu.VMEM((2, 2, EB), jnp.int32),
        pltpu.VMEM((2, 2, EB), jnp.int32),
        pltpu.VMEM((2, EB, DQP), jnp.float32),
        pltpu.VMEM((2, EB, DQP), jnp.float32),
        pltpu.SemaphoreType.DMA,
        pltpu.SemaphoreType.DMA,
        pltpu.SemaphoreType.DMA,
        pltpu.SemaphoreType.DMA,
    ],
    compiler_params=pltpu.CompilerParams(use_tc_tiling_on_sc=False),
)


def _mlp_body(h_ref, w1_ref, b1_ref, w2_ref, b2_ref, out_ref):
    hcat = jnp.concatenate([h_ref[q] for q in range(NQ)], axis=1)
    z = lax.dot_general(hcat, w1_ref[...], (((1,), (0,)), ((), ())),
                        preferred_element_type=jnp.float32,
                        precision=lax.Precision.HIGHEST)
    z = jnp.maximum(z + b1_ref[...], 0.0)
    w2s = jnp.sum(w2_ref[...], axis=1)
    out_ref[...] = (jnp.sum(z * w2s[None, :], axis=1, keepdims=True)
                    + jnp.sum(b2_ref[...]))


_mlp = pl.pallas_call(
    _mlp_body,
    grid=(N_NODES // RB,),
    in_specs=[
        pl.BlockSpec((NQ, RB, DQP), lambda i: (0, i, 0)),
        pl.BlockSpec((DOUT, HIDDEN), lambda i: (0, 0)),
        pl.BlockSpec((1, HIDDEN), lambda i: (0, 0)),
        pl.BlockSpec((HIDDEN, HIDDEN), lambda i: (0, 0)),
        pl.BlockSpec((1, HIDDEN), lambda i: (0, 0)),
    ],
    out_specs=pl.BlockSpec((RB, 1), lambda i: (i, 0)),
    out_shape=jax.ShapeDtypeStruct((N_NODES, 1), jnp.float32),
)


def _gather_body(s_hbm, dst_hbm, out_hbm, s_v, dst_v, out_v):
    c = lax.axis_index("c")
    s = lax.axis_index("s")
    w = s * N_SC + c
    pltpu.sync_copy(s_hbm, s_v)
    nb = BLK2_PER_W + jnp.where(w < NBLK2 % NW, 1, 0)

    def blk(i, carry):
        off = pl.multiple_of((w + i * NW) * EB2, EB2)
        pltpu.sync_copy(dst_hbm.at[pl.ds(off, EB2)], dst_v)

        def inner(j, c2):
            idx = dst_v[pl.ds(j * 16, 16)]
            out_v[pl.ds(j * 16, 16)] = plsc.load_gather(s_v, [idx])
            return c2

        lax.fori_loop(0, EB2 // 16, inner, 0)
        pltpu.sync_copy(out_v, out_hbm.at[pl.ds(off, EB2)])
        return carry

    lax.fori_loop(0, nb, blk, 0)


_gather = pl.kernel(
    _gather_body,
    out_type=jax.ShapeDtypeStruct((N_EDGES,), jnp.float32),
    mesh=plsc.VectorSubcoreMesh(core_axis_name="c", subcore_axis_name="s"),
    scratch_types=[
        pltpu.VMEM((N_NODES,), jnp.float32),
        pltpu.VMEM((EB2,), jnp.int32),
        pltpu.VMEM((EB2,), jnp.float32),
    ],
    compiler_params=pltpu.CompilerParams(needs_layout_passes=False),
)


def kernel(x, edge_index, pos_embeddings, W1, b1, W2, b2):
    xf = x.reshape(N_NODES, D_IN)
    ei = edge_index.astype(jnp.int32)
    srcp = jnp.concatenate([ei[0], jnp.zeros((E_PAD - N_EDGES,), jnp.int32)])
    dstp = jnp.concatenate(
        [ei[1], jnp.full((E_PAD - N_EDGES,), N_NODES, jnp.int32)])
    edges = jnp.stack(
        [srcp.reshape(NBLK, EB), dstp.reshape(NBLK, EB)], axis=1)
    xqs = jnp.pad(xf.reshape(N_NODES, NQ, DQ),
                  ((0, 0), (0, 0), (0, DQP - DQ))).transpose(1, 0, 2)
    h4 = _agg(xqs, edges)
    W1p = jnp.pad(W1.reshape(NQ, DQ, HIDDEN),
                  ((0, 0), (0, DQP - DQ), (0, 0))).reshape(DOUT, HIDDEN)
    s = _mlp(h4, W1p, b1.reshape(1, HIDDEN), W2,
             b2.reshape(1, HIDDEN)).reshape(N_NODES)
    return _gather(s, ei[1])


# speedup vs baseline: 1.9477x; 1.2633x over previous
import jax
import jax.numpy as jnp
from jax import lax
from jax.experimental import pallas as pl
from jax.experimental.pallas import tpu as pltpu
from jax.experimental.pallas import tpu_sc as plsc

N_NODES = 50000
N_EDGES = 800000
D_IN = 100
HIDDEN = 256
NQ = 4
DQ = 25
DQP = 32
DOUT = NQ * DQP
N_SC = 2
N_TILES = 16
STRIPE = 3200
LAST_STRIPE = N_NODES - (N_TILES - 1) * STRIPE
EB = 128
BLK_PER_TILE = 392
NBLK = BLK_PER_TILE * N_TILES
E_PAD = NBLK * EB
ACC_ROWS = 50048
NGRP = BLK_PER_TILE // 4
EB2 = 800
NBLK2 = N_EDGES // EB2
NW = N_SC * N_TILES
BLK2_PER_W = NBLK2 // NW
RB = 1000


def _stripe_copy(s, read, write):
    off = pl.multiple_of(s * STRIPE, STRIPE)

    @pl.when(s < N_TILES - 1)
    def _main():
        pltpu.sync_copy(read(pl.ds(off, STRIPE)), write(pl.ds(off, STRIPE)))

    @pl.when(s == N_TILES - 1)
    def _last():
        base = (N_TILES - 1) * STRIPE
        pltpu.sync_copy(read(pl.ds(base, LAST_STRIPE)),
                        write(pl.ds(base, LAST_STRIPE)))


def _agg_body(xq_hbm, edges_hbm, out_hbm, acc_sh, idx_a, idx_b, rows_a,
              rows_b, isem_a, isem_b, gsem_a, gsem_b):
    c = lax.axis_index("c")
    s = lax.axis_index("s")

    for p in range(2):
        for cc in range(N_SC):
            q = 2 * cc + p

            @pl.when(c == cc)
            def _init(q=q):
                _stripe_copy(s, lambda d: xq_hbm.at[q, d],
                             lambda d: acc_sh.at[d])

        plsc.subcore_barrier()

        for cc in range(N_SC):
            q = 2 * cc + p

            @pl.when(c == cc)
            def _edges(q=q):
                table = xq_hbm.at[q]
                base = pl.multiple_of(s * BLK_PER_TILE, BLK_PER_TILE)

                def idx_slice(off):
                    return edges_hbm.at[pl.ds(pl.multiple_of(off, 2), 2)]

                def gather(j, idx, rows, sem):
                    return pltpu.async_copy(table.at[idx.at[j, 0]],
                                            rows.at[j], sem)

                def gather_wait(j, idx, rows, sem):
                    pltpu.make_async_copy(table.at[idx.at[j, 0]],
                                          rows.at[j], sem).wait()

                def scat(j, idx, rows):
                    pltpu.sync_copy(rows.at[j], acc_sh.at[idx.at[j, 1]],
                                    add=True)

                pltpu.async_copy(idx_slice(base), idx_a, isem_a)
                pltpu.async_copy(idx_slice(base + 2), idx_b, isem_b)
                pltpu.make_async_copy(idx_slice(base), idx_a, isem_a).wait()
                gather(0, idx_a, rows_a, gsem_a)
                gather(1, idx_a, rows_a, gsem_a)

                def grp(g, carry):
                    koff = pl.multiple_of(base + 4 * g, 2)
                    gather_wait(0, idx_a, rows_a, gsem_a)
                    gather_wait(1, idx_a, rows_a, gsem_a)
                    pltpu.make_async_copy(idx_slice(koff + 2), idx_b,
                                          isem_b).wait()
                    hb0 = gather(0, idx_b, rows_b, gsem_b)
                    hb1 = gather(1, idx_b, rows_b, gsem_b)
                    scat(0, idx_a, rows_a)
                    scat(1, idx_a, rows_a)
                    hla = pltpu.async_copy(idx_slice(koff + 4), idx_a, isem_a)
                    hb0.wait()
                    hb1.wait()
                    hla.wait()
                    gather(0, idx_a, rows_a, gsem_a)
                    gather(1, idx_a, rows_a, gsem_a)
                    scat(0, idx_b, rows_b)
                    scat(1, idx_b, rows_b)
                    pltpu.async_copy(idx_slice(koff + 6), idx_b, isem_b)
                    return carry

                lax.fori_loop(0, NGRP - 1, grp, 0)

                gather_wait(0, idx_a, rows_a, gsem_a)
                gather_wait(1, idx_a, rows_a, gsem_a)
                pltpu.make_async_copy(idx_slice(base + BLK_PER_TILE - 2),
                                      idx_b, isem_b).wait()
                hb0 = gather(0, idx_b, rows_b, gsem_b)
                hb1 = gather(1, idx_b, rows_b, gsem_b)
                scat(0, idx_a, rows_a)
                scat(1, idx_a, rows_a)
                hb0.wait()
                hb1.wait()
                scat(0, idx_b, rows_b)
                scat(1, idx_b, rows_b)

        plsc.subcore_barrier()

        for cc in range(N_SC):
            q = 2 * cc + p

            @pl.when(c == cc)
            def _flush(q=q):
                _stripe_copy(s, lambda d: acc_sh.at[d],
                             lambda d: out_hbm.at[q, d])

        plsc.subcore_barrier()


_agg = pl.kernel(
    _agg_body,
    out_type=jax.ShapeDtypeStruct((NQ, N_NODES, DQP), jnp.float32),
    mesh=plsc.VectorSubcoreMesh(core_axis_name="c", subcore_axis_name="s"),
    scratch_types=[
        pltpu.VMEM_SHARED((ACC_ROWS, DQP), jnp.float32),
        pltpu.VMEM((2, 2, EB), jnp.int32),
        pltpu.VMEM((2, 2, EB), jnp.int32),
        pltpu.VMEM((2, EB, DQP), jnp.float32),
        pltpu.VMEM((2, EB, DQP), jnp.float32),
        pltpu.SemaphoreType.DMA,
        pltpu.SemaphoreType.DMA,
        pltpu.SemaphoreType.DMA,
        pltpu.SemaphoreType.DMA,
    ],
    compiler_params=pltpu.CompilerParams(use_tc_tiling_on_sc=False),
)


def _mlp_body(h_ref, w1_ref, b1_ref, w2_ref, b2_ref, out_ref):
    hcat = jnp.concatenate([h_ref[q] for q in range(NQ)], axis=1)
    z = lax.dot_general(hcat, w1_ref[...], (((1,), (0,)), ((), ())),
                        preferred_element_type=jnp.float32,
                        precision=lax.Precision.HIGHEST)
    z = jnp.maximum(z + b1_ref[...], 0.0)
    w2s = jnp.sum(w2_ref[...], axis=1)
    out_ref[...] = (jnp.sum(z * w2s[None, :], axis=1, keepdims=True)
                    + jnp.sum(b2_ref[...]))


_mlp = pl.pallas_call(
    _mlp_body,
    grid=(N_NODES // RB,),
    in_specs=[
        pl.BlockSpec((NQ, RB, DQP), lambda i: (0, i, 0)),
        pl.BlockSpec((DOUT, HIDDEN), lambda i: (0, 0)),
        pl.BlockSpec((1, HIDDEN), lambda i: (0, 0)),
        pl.BlockSpec((HIDDEN, HIDDEN), lambda i: (0, 0)),
        pl.BlockSpec((1, HIDDEN), lambda i: (0, 0)),
    ],
    out_specs=pl.BlockSpec((RB, 1), lambda i: (i, 0)),
    out_shape=jax.ShapeDtypeStruct((N_NODES, 1), jnp.float32),
)


def _gather_body(s_hbm, dst_hbm, out_hbm, s_v, dst_v, out_v):
    c = lax.axis_index("c")
    s = lax.axis_index("s")
    w = s * N_SC + c
    pltpu.sync_copy(s_hbm, s_v)
    nb = BLK2_PER_W + jnp.where(w < NBLK2 % NW, 1, 0)

    def blk(i, carry):
        off = pl.multiple_of((w + i * NW) * EB2, EB2)
        pltpu.sync_copy(dst_hbm.at[pl.ds(off, EB2)], dst_v)

        def inner(j, c2):
            idx = dst_v[pl.ds(j * 16, 16)]
            out_v[pl.ds(j * 16, 16)] = plsc.load_gather(s_v, [idx])
            return c2

        lax.fori_loop(0, EB2 // 16, inner, 0)
        pltpu.sync_copy(out_v, out_hbm.at[pl.ds(off, EB2)])
        return carry

    lax.fori_loop(0, nb, blk, 0)


_gather = pl.kernel(
    _gather_body,
    out_type=jax.ShapeDtypeStruct((N_EDGES,), jnp.float32),
    mesh=plsc.VectorSubcoreMesh(core_axis_name="c", subcore_axis_name="s"),
    scratch_types=[
        pltpu.VMEM((N_NODES,), jnp.float32),
        pltpu.VMEM((EB2,), jnp.int32),
        pltpu.VMEM((EB2,), jnp.float32),
    ],
    compiler_params=pltpu.CompilerParams(needs_layout_passes=False),
)


def kernel(x, edge_index, pos_embeddings, W1, b1, W2, b2):
    perm = (jnp.arange(D_IN)[:, None]
            == (jnp.arange(NQ * DQP) % DQP
                + DQ * (jnp.arange(NQ * DQP) // DQP))[None, :]
            ).astype(jnp.float32)
    xf = x.reshape(N_NODES, D_IN)
    ei = edge_index.astype(jnp.int32)
    srcp = jnp.concatenate([ei[0], jnp.zeros((E_PAD - N_EDGES,), jnp.int32)])
    dstp = jnp.concatenate(
        [ei[1], jnp.full((E_PAD - N_EDGES,), N_NODES, jnp.int32)])
    edges = jnp.stack(
        [srcp.reshape(NBLK, EB), dstp.reshape(NBLK, EB)], axis=1)
    xp = lax.dot_general(xf, perm, (((1,), (0,)), ((), ())),
                         preferred_element_type=jnp.float32,
                         precision=lax.Precision.HIGHEST)
    xqs = xp.reshape(N_NODES, NQ, DQP).transpose(1, 0, 2)
    h4 = _agg(xqs, edges)
    W1p = jnp.pad(W1.reshape(NQ, DQ, HIDDEN),
                  ((0, 0), (0, DQP - DQ), (0, 0))).reshape(DOUT, HIDDEN)
    s = _mlp(h4, W1p, b1.reshape(1, HIDDEN), W2,
             b2.reshape(1, HIDDEN)).reshape(N_NODES)
    return _gather(s, ei[1])
